# baseline (speedup 1.0000x reference)
import jax
import jax.numpy as jnp
from jax import lax
from jax.experimental import pallas as pl
from jax.experimental.pallas import tpu as pltpu
from jax.experimental.pallas import tpu_sc as plsc

N = 10000
E = 160000
D_IN = 256
D_HID = 512
D_OUT = 256
HF = 128
NC, NS = 2, 16
NW = NC * NS
K = 125
EPT = E // NS
NCH = EPT // K
NP = 10240
RPT = NP // NS
DW = 16

_MESH = dict(core_axis_name="c", subcore_axis_name="s", num_cores=NC,
             num_subcores=NS)
_NOTILE = pltpu.CompilerParams(use_tc_tiling_on_sc=False)


def _deg_body(edges4, ones_hbm, zeros_hbm, deg_out, idx_v, ones_v, deg_sh):
    c = lax.axis_index("c")
    s = lax.axis_index("s")
    w = c * NS + s
    pltpu.sync_copy(zeros_hbm, deg_sh.at[pl.ds(s * RPT, RPT)])
    pltpu.sync_copy(ones_hbm, ones_v)
    pltpu.sync_copy(edges4.at[w], idx_v)
    plsc.subcore_barrier()

    def chunk(j, carry):
        pltpu.sync_copy(ones_v, deg_sh.at[idx_v.at[j]], add=True)
        return carry

    lax.fori_loop(0, NCH, chunk, 0)
    plsc.subcore_barrier()
    pltpu.sync_copy(deg_sh.at[pl.ds(s * RPT, RPT)],
                    deg_out.at[pl.ds(c * NP + s * RPT, RPT)])


def _agg_body(hh, hsrc, dst3, zeros_hbm, out, sidx, didx, msg0, msg1,
              acc_sh, sem0, sem1):
    c = lax.axis_index("c")
    s = lax.axis_index("s")
    pltpu.sync_copy(zeros_hbm, acc_sh.at[pl.ds(s * RPT, RPT)])
    pltpu.sync_copy(dst3.at[s], didx)
    pltpu.sync_copy(hsrc.at[c * NS + s], sidx)
    plsc.subcore_barrier()

    pltpu.async_copy(hh.at[sidx.at[0]], msg0, sem0)

    def pair(jj, carry):
        j = jj * 2
        pltpu.async_copy(hh.at[sidx.at[j + 1]], msg1, sem1)
        pltpu.make_async_copy(hh.at[sidx.at[j]], msg0, sem0).wait()
        pltpu.sync_copy(msg0, acc_sh.at[didx.at[j]], add=True)

        @pl.when(j + 2 < NCH)
        def _():
            pltpu.async_copy(hh.at[sidx.at[j + 2]], msg0, sem0)

        pltpu.make_async_copy(hh.at[sidx.at[j + 1]], msg1, sem1).wait()
        pltpu.sync_copy(msg1, acc_sh.at[didx.at[j + 1]], add=True)
        return carry

    lax.fori_loop(0, NCH // 2, pair, 0)
    plsc.subcore_barrier()
    pltpu.sync_copy(acc_sh.at[pl.ds(s * RPT, RPT)],
                    out.at[pl.ds(c * NP + s * RPT, RPT)])


def _sc_degrees(edges4, ones16, zeros16):
    f = pl.kernel(
        _deg_body,
        out_type=jax.ShapeDtypeStruct((2 * NP, DW), jnp.float32),
        mesh=plsc.VectorSubcoreMesh(**_MESH),
        scratch_types=[
            pltpu.VMEM((NCH, K), jnp.int32),
            pltpu.VMEM((K, DW), jnp.float32),
            pltpu.VMEM_SHARED((NP, DW), jnp.float32),
        ],
        compiler_params=_NOTILE,
    )
    return f(edges4, ones16, zeros16)


def _sc_aggregate(hh, hsrc, dst3, zerosH):
    f = pl.kernel(
        _agg_body,
        out_type=jax.ShapeDtypeStruct((2 * NP, HF), jnp.bfloat16),
        mesh=plsc.VectorSubcoreMesh(**_MESH),
        scratch_types=[
            pltpu.VMEM((NCH, K), jnp.int32),
            pltpu.VMEM((NCH, K), jnp.int32),
            pltpu.VMEM((K, HF), jnp.bfloat16),
            pltpu.VMEM((K, HF), jnp.bfloat16),
            pltpu.VMEM_SHARED((NP, HF), jnp.bfloat16),
            pltpu.SemaphoreType.DMA,
            pltpu.SemaphoreType.DMA,
        ],
        compiler_params=_NOTILE,
    )
    return f(hh, hsrc, dst3, zerosH)


BN = 640


def _norm(deg_ref):
    d = deg_ref[:, 0:1]
    return jnp.where(d > 0, lax.rsqrt(jnp.maximum(d, 1e-12)), 0.0)


def _prescale_body(ds_ref, x_ref, hh_ref):
    h = (x_ref[...] * _norm(ds_ref)).astype(jnp.bfloat16)
    hh_ref[0] = h[:, :HF]
    hh_ref[1] = h[:, HF:]


def _tc_prescale(deg, x):
    return pl.pallas_call(
        _prescale_body,
        grid=(NP // BN,),
        in_specs=[pl.BlockSpec((BN, DW), lambda i: (i, 0)),
                  pl.BlockSpec((BN, D_IN), lambda i: (i, 0))],
        out_specs=pl.BlockSpec((2, BN, HF), lambda i: (0, i, 0)),
        out_shape=jax.ShapeDtypeStruct((2, NP, HF), jnp.bfloat16),
    )(deg, x)


def _mid_body(dd_ref, ds_ref, agg_ref, w1_ref, b1_ref, w2_ref, zh_ref):
    i = pl.program_id(0)
    a = agg_ref[pl.ds(i * BN, BN), :]
    b = agg_ref[pl.ds(NP + i * BN, BN), :]
    nd = _norm(dd_ref).astype(jnp.bfloat16)
    agg = jnp.concatenate([a, b], axis=1) * nd
    t = jnp.dot(agg, w1_ref[...], preferred_element_type=jnp.float32)
    t = jnp.maximum(t + b1_ref[...], 0.0)
    tb = (t * _norm(ds_ref)).astype(jnp.bfloat16)
    z = (jnp.dot(tb, w2_ref[...],
                 preferred_element_type=jnp.float32)).astype(jnp.bfloat16)
    zh_ref[0] = z[:, :HF]
    zh_ref[1] = z[:, HF:]


def _tc_mid(deg, agg1, w1, b1, w2):
    nb = NP // BN
    return pl.pallas_call(
        _mid_body,
        grid=(nb,),
        in_specs=[pl.BlockSpec((BN, DW), lambda i: (nb + i, 0)),
                  pl.BlockSpec((BN, DW), lambda i: (i, 0)),
                  pl.BlockSpec((2 * NP, HF), lambda i: (0, 0)),
                  pl.BlockSpec((D_IN, D_HID), lambda i: (0, 0)),
                  pl.BlockSpec((1, D_HID), lambda i: (0, 0)),
                  pl.BlockSpec((D_HID, D_OUT), lambda i: (0, 0))],
        out_specs=pl.BlockSpec((2, BN, HF), lambda i: (0, i, 0)),
        out_shape=jax.ShapeDtypeStruct((2, NP, HF), jnp.bfloat16),
    )(deg, deg, agg1, w1.astype(jnp.bfloat16), b1, w2.astype(jnp.bfloat16))


def _final_body(dd_ref, agg_ref, b2_ref, wp_ref, bp_ref, out_ref):
    i = pl.program_id(0)
    a = agg_ref[pl.ds(i * BN, BN), :]
    b = agg_ref[pl.ds(NP + i * BN, BN), :]
    agg = jnp.concatenate([a, b], axis=1).astype(jnp.float32) * _norm(dd_ref)
    x2 = jnp.maximum(agg + b2_ref[...], 0.0)
    logits = jnp.dot(x2, wp_ref[...], preferred_element_type=jnp.float32)
    out_ref[...] = jax.nn.sigmoid(logits + bp_ref[0, 0:1])


def _tc_final(deg, agg2, b2, wp, bp):
    nb = NP // BN
    return pl.pallas_call(
        _final_body,
        grid=(nb,),
        in_specs=[pl.BlockSpec((BN, DW), lambda i: (nb + i, 0)),
                  pl.BlockSpec((2 * NP, HF), lambda i: (0, 0)),
                  pl.BlockSpec((1, D_OUT), lambda i: (0, 0)),
                  pl.BlockSpec((D_OUT, 1), lambda i: (0, 0)),
                  pl.BlockSpec((1, 1), lambda i: (0, 0))],
        out_specs=pl.BlockSpec((BN, 1), lambda i: (i, 0)),
        out_shape=jax.ShapeDtypeStruct((NP, 1), jnp.float32),
    )(deg, agg2, b2, wp, bp)


def kernel(features, edge_index, edge_types, W1, b1, W2, b2, Wp, bp):
    src = edge_index[0]
    dst = edge_index[1]
    dst3 = dst.reshape(NS, NCH, K)
    edges4 = jnp.concatenate([src, dst]).reshape(NW, NCH, K)
    hsrc = (src.reshape(1, NS, NCH, K)
            + (jnp.arange(2, dtype=jnp.int32) * NP).reshape(2, 1, 1, 1)
            ).reshape(2 * NS, NCH, K)
    ones16 = jnp.ones((K, DW), jnp.float32)
    zeros16 = jnp.zeros((RPT, DW), jnp.float32)
    zerosH = jnp.zeros((RPT, HF), jnp.bfloat16)

    deg = _sc_degrees(edges4, ones16, zeros16)
    hh = _tc_prescale(deg, features).reshape(2 * NP, HF)
    agg1 = _sc_aggregate(hh, hsrc, dst3, zerosH)
    zh = _tc_mid(deg, agg1, W1, b1.reshape(1, D_HID), W2).reshape(2 * NP, HF)
    agg2 = _sc_aggregate(zh, hsrc, dst3, zerosH)
    out = _tc_final(deg, agg2, b2.reshape(1, D_OUT), Wp, bp.reshape(1, 1))
    return out[:N, 0]


# speedup vs baseline: 1.0246x; 1.0246x over previous
import jax
import jax.numpy as jnp
from jax import lax
from jax.experimental import pallas as pl
from jax.experimental.pallas import tpu as pltpu
from jax.experimental.pallas import tpu_sc as plsc

N = 10000
E = 160000
D_IN = 256
D_HID = 512
D_OUT = 256
HF = 128
NC, NS = 2, 16
NW = NC * NS
K = 125
EPT = E // NS
NCH = EPT // K
NP = 10240
RPT = NP // NS
DW = 16

_MESH = dict(core_axis_name="c", subcore_axis_name="s", num_cores=NC,
             num_subcores=NS)
_NOTILE = pltpu.CompilerParams(use_tc_tiling_on_sc=False)


def _deg_body(edges4, ones_hbm, zeros_hbm, deg_out, idx_v, ones_v, deg_sh):
    c = lax.axis_index("c")
    s = lax.axis_index("s")
    w = c * NS + s
    pltpu.sync_copy(zeros_hbm, deg_sh.at[pl.ds(s * RPT, RPT)])
    pltpu.sync_copy(ones_hbm, ones_v)
    pltpu.sync_copy(edges4.at[w], idx_v)
    plsc.subcore_barrier()

    def chunk(j, carry):
        pltpu.sync_copy(ones_v, deg_sh.at[idx_v.at[j]], add=True)
        return carry

    lax.fori_loop(0, NCH, chunk, 0)
    plsc.subcore_barrier()
    pltpu.sync_copy(deg_sh.at[pl.ds(s * RPT, RPT)],
                    deg_out.at[pl.ds(c * NP + s * RPT, RPT)])


def _agg_body(hh, er, zeros_hbm, out, sidx, didx, msg0, msg1,
              acc_sh, sem0, sem1):
    c = lax.axis_index("c")
    s = lax.axis_index("s")
    ht = hh.at[pl.ds(c * NP, NP)]
    pltpu.sync_copy(zeros_hbm, acc_sh.at[pl.ds(s * RPT, RPT)])
    pltpu.sync_copy(er.at[NS + s], didx)
    pltpu.sync_copy(er.at[s], sidx)
    plsc.subcore_barrier()

    pltpu.async_copy(ht.at[sidx.at[0]], msg0, sem0)

    def pair(jj, carry):
        j = jj * 2
        pltpu.async_copy(ht.at[sidx.at[j + 1]], msg1, sem1)
        pltpu.make_async_copy(ht.at[sidx.at[j]], msg0, sem0).wait()
        pltpu.sync_copy(msg0, acc_sh.at[didx.at[j]], add=True)

        @pl.when(j + 2 < NCH)
        def _():
            pltpu.async_copy(ht.at[sidx.at[j + 2]], msg0, sem0)

        pltpu.make_async_copy(ht.at[sidx.at[j + 1]], msg1, sem1).wait()
        pltpu.sync_copy(msg1, acc_sh.at[didx.at[j + 1]], add=True)
        return carry

    lax.fori_loop(0, NCH // 2, pair, 0)
    plsc.subcore_barrier()
    pltpu.sync_copy(acc_sh.at[pl.ds(s * RPT, RPT)],
                    out.at[pl.ds(c * NP + s * RPT, RPT)])


def _sc_degrees(edges4, ones16, zeros16):
    f = pl.kernel(
        _deg_body,
        out_type=jax.ShapeDtypeStruct((2 * NP, DW), jnp.float32),
        mesh=plsc.VectorSubcoreMesh(**_MESH),
        scratch_types=[
            pltpu.VMEM((NCH, K), jnp.int32),
            pltpu.VMEM((K, DW), jnp.float32),
            pltpu.VMEM_SHARED((NP, DW), jnp.float32),
        ],
        compiler_params=_NOTILE,
    )
    return f(edges4, ones16, zeros16)


def _sc_aggregate(hh, er, zerosH):
    f = pl.kernel(
        _agg_body,
        out_type=jax.ShapeDtypeStruct((2 * NP, HF), jnp.bfloat16),
        mesh=plsc.VectorSubcoreMesh(**_MESH),
        scratch_types=[
            pltpu.VMEM((NCH, K), jnp.int32),
            pltpu.VMEM((NCH, K), jnp.int32),
            pltpu.VMEM((K, HF), jnp.bfloat16),
            pltpu.VMEM((K, HF), jnp.bfloat16),
            pltpu.VMEM_SHARED((NP, HF), jnp.bfloat16),
            pltpu.SemaphoreType.DMA,
            pltpu.SemaphoreType.DMA,
        ],
        compiler_params=_NOTILE,
    )
    return f(hh, er, zerosH)


BN = 640


def _norm(deg_ref):
    d = deg_ref[:, 0:1]
    return jnp.where(d > 0, lax.rsqrt(jnp.maximum(d, 1e-12)), 0.0)


def _prescale_body(ds_ref, x_ref, hh_ref):
    h = (x_ref[...] * _norm(ds_ref)).astype(jnp.bfloat16)
    hh_ref[0] = h[:, :HF]
    hh_ref[1] = h[:, HF:]


def _tc_prescale(deg, x):
    return pl.pallas_call(
        _prescale_body,
        grid=(NP // BN,),
        in_specs=[pl.BlockSpec((BN, DW), lambda i: (i, 0)),
                  pl.BlockSpec((BN, D_IN), lambda i: (i, 0))],
        out_specs=pl.BlockSpec((2, BN, HF), lambda i: (0, i, 0)),
        out_shape=jax.ShapeDtypeStruct((2, NP, HF), jnp.bfloat16),
    )(deg, x)


def _mid_body(dd_ref, ds_ref, agg_ref, w1_ref, b1_ref, w2_ref, zh_ref):
    i = pl.program_id(0)
    a = agg_ref[pl.ds(i * BN, BN), :]
    b = agg_ref[pl.ds(NP + i * BN, BN), :]
    nd = _norm(dd_ref).astype(jnp.bfloat16)
    agg = jnp.concatenate([a, b], axis=1) * nd
    t = jnp.dot(agg, w1_ref[...], preferred_element_type=jnp.float32)
    t = jnp.maximum(t + b1_ref[...], 0.0)
    tb = (t * _norm(ds_ref)).astype(jnp.bfloat16)
    z = (jnp.dot(tb, w2_ref[...],
                 preferred_element_type=jnp.float32)).astype(jnp.bfloat16)
    zh_ref[0] = z[:, :HF]
    zh_ref[1] = z[:, HF:]


def _tc_mid(deg, agg1, w1, b1, w2):
    nb = NP // BN
    return pl.pallas_call(
        _mid_body,
        grid=(nb,),
        in_specs=[pl.BlockSpec((BN, DW), lambda i: (nb + i, 0)),
                  pl.BlockSpec((BN, DW), lambda i: (i, 0)),
                  pl.BlockSpec((2 * NP, HF), lambda i: (0, 0)),
                  pl.BlockSpec((D_IN, D_HID), lambda i: (0, 0)),
                  pl.BlockSpec((1, D_HID), lambda i: (0, 0)),
                  pl.BlockSpec((D_HID, D_OUT), lambda i: (0, 0))],
        out_specs=pl.BlockSpec((2, BN, HF), lambda i: (0, i, 0)),
        out_shape=jax.ShapeDtypeStruct((2, NP, HF), jnp.bfloat16),
    )(deg, deg, agg1, w1.astype(jnp.bfloat16), b1, w2.astype(jnp.bfloat16))


def _final_body(dd_ref, agg_ref, b2_ref, wp_ref, bp_ref, out_ref):
    i = pl.program_id(0)
    a = agg_ref[pl.ds(i * BN, BN), :]
    b = agg_ref[pl.ds(NP + i * BN, BN), :]
    agg = jnp.concatenate([a, b], axis=1).astype(jnp.float32) * _norm(dd_ref)
    x2 = jnp.maximum(agg + b2_ref[...], 0.0)
    logits = jnp.dot(x2, wp_ref[...], preferred_element_type=jnp.float32)
    out_ref[...] = jax.nn.sigmoid(logits + bp_ref[0, 0:1])


def _tc_final(deg, agg2, b2, wp, bp):
    nb = NP // BN
    return pl.pallas_call(
        _final_body,
        grid=(nb,),
        in_specs=[pl.BlockSpec((BN, DW), lambda i: (nb + i, 0)),
                  pl.BlockSpec((2 * NP, HF), lambda i: (0, 0)),
                  pl.BlockSpec((1, D_OUT), lambda i: (0, 0)),
                  pl.BlockSpec((D_OUT, 1), lambda i: (0, 0)),
                  pl.BlockSpec((1, 1), lambda i: (0, 0))],
        out_specs=pl.BlockSpec((BN, 1), lambda i: (i, 0)),
        out_shape=jax.ShapeDtypeStruct((NP, 1), jnp.float32),
    )(deg, agg2, b2, wp, bp)


def kernel(features, edge_index, edge_types, W1, b1, W2, b2, Wp, bp):
    er = edge_index.reshape(NW, NCH, K)
    ones16 = jnp.ones((K, DW), jnp.float32)
    zeros16 = jnp.zeros((RPT, DW), jnp.float32)
    zerosH = jnp.zeros((RPT, HF), jnp.bfloat16)

    deg = _sc_degrees(er, ones16, zeros16)
    hh = _tc_prescale(deg, features).reshape(2 * NP, HF)
    agg1 = _sc_aggregate(hh, er, zerosH)
    zh = _tc_mid(deg, agg1, W1, b1.reshape(1, D_HID), W2).reshape(2 * NP, HF)
    agg2 = _sc_aggregate(zh, er, zerosH)
    out = _tc_final(deg, agg2, b2.reshape(1, D_OUT), Wp, bp.reshape(1, 1))
    return out[:N, 0]


# speedup vs baseline: 1.1375x; 1.1102x over previous
import jax
import jax.numpy as jnp
from jax import lax
from jax.experimental import pallas as pl
from jax.experimental.pallas import tpu as pltpu
from jax.experimental.pallas import tpu_sc as plsc

N = 10000
E = 160000
D_IN = 256
D_HID = 512
D_OUT = 256
HF = 128
NC, NS = 2, 16
NW = NC * NS
K = 125
EPT = E // NS
NCH = EPT // K
NP = 10240
RPT = NP // NS
DW = 16

_MESH = dict(core_axis_name="c", subcore_axis_name="s", num_cores=NC,
             num_subcores=NS)
_NOTILE = pltpu.CompilerParams(use_tc_tiling_on_sc=False)


def _deg_body(edges4, ones_hbm, zeros_hbm, deg_out, idx_v, ones_v, deg_sh):
    c = lax.axis_index("c")
    s = lax.axis_index("s")
    w = c * NS + s
    pltpu.sync_copy(zeros_hbm, deg_sh.at[pl.ds(s * RPT, RPT)])
    pltpu.sync_copy(ones_hbm, ones_v)
    pltpu.sync_copy(edges4.at[w], idx_v)
    plsc.subcore_barrier()

    def chunk(j, carry):
        pltpu.sync_copy(ones_v, deg_sh.at[idx_v.at[j]], add=True)
        return carry

    lax.fori_loop(0, NCH, chunk, 0)
    plsc.subcore_barrier()
    pltpu.sync_copy(deg_sh.at[pl.ds(s * RPT, RPT)],
                    deg_out.at[pl.ds(c * NP + s * RPT, RPT)])


def _agg_body(hh, er, zeros_hbm, out, sidx, didx, msg0, msg1, msg2, msg3,
              acc_sh, sem0, sem1, sem2, sem3):
    c = lax.axis_index("c")
    s = lax.axis_index("s")
    ht = hh.at[pl.ds(c * NP, NP)]
    bufs = (msg0, msg1, msg2, msg3)
    sems = (sem0, sem1, sem2, sem3)
    pltpu.sync_copy(zeros_hbm, acc_sh.at[pl.ds(s * RPT, RPT)])
    pltpu.sync_copy(er.at[NS + s], didx)
    pltpu.sync_copy(er.at[s], sidx)
    plsc.subcore_barrier()

    for u in range(4):
        pltpu.async_copy(ht.at[sidx.at[u]], bufs[u], sems[u])

    def quad(jj, carry):
        t0 = jj * 4
        for u in range(4):
            t = t0 + u
            pltpu.make_async_copy(ht.at[sidx.at[t]], bufs[u], sems[u]).wait()
            pltpu.sync_copy(bufs[u], acc_sh.at[didx.at[t]], add=True)

            @pl.when(t + 4 < NCH)
            def _():
                pltpu.async_copy(ht.at[sidx.at[t + 4]], bufs[u], sems[u])

        return carry

    lax.fori_loop(0, NCH // 4, quad, 0)
    plsc.subcore_barrier()
    pltpu.sync_copy(acc_sh.at[pl.ds(s * RPT, RPT)],
                    out.at[pl.ds(c * NP + s * RPT, RPT)])


def _sc_degrees(edges4, ones16, zeros16):
    f = pl.kernel(
        _deg_body,
        out_type=jax.ShapeDtypeStruct((2 * NP, DW), jnp.float32),
        mesh=plsc.VectorSubcoreMesh(**_MESH),
        scratch_types=[
            pltpu.VMEM((NCH, K), jnp.int32),
            pltpu.VMEM((K, DW), jnp.float32),
            pltpu.VMEM_SHARED((NP, DW), jnp.float32),
        ],
        compiler_params=_NOTILE,
    )
    return f(edges4, ones16, zeros16)


def _sc_aggregate(hh, er, zerosH):
    f = pl.kernel(
        _agg_body,
        out_type=jax.ShapeDtypeStruct((2 * NP, HF), jnp.bfloat16),
        mesh=plsc.VectorSubcoreMesh(**_MESH),
        scratch_types=[
            pltpu.VMEM((NCH, K), jnp.int32),
            pltpu.VMEM((NCH, K), jnp.int32),
            pltpu.VMEM((K, HF), jnp.bfloat16),
            pltpu.VMEM((K, HF), jnp.bfloat16),
            pltpu.VMEM((K, HF), jnp.bfloat16),
            pltpu.VMEM((K, HF), jnp.bfloat16),
            pltpu.VMEM_SHARED((NP, HF), jnp.bfloat16),
            pltpu.SemaphoreType.DMA,
            pltpu.SemaphoreType.DMA,
            pltpu.SemaphoreType.DMA,
            pltpu.SemaphoreType.DMA,
        ],
        compiler_params=_NOTILE,
    )
    return f(hh, er, zerosH)


BN = 640


def _norm(deg_ref):
    d = deg_ref[:, 0:1]
    return jnp.where(d > 0, lax.rsqrt(jnp.maximum(d, 1e-12)), 0.0)


def _prescale_body(ds_ref, x_ref, hh_ref):
    h = (x_ref[...] * _norm(ds_ref)).astype(jnp.bfloat16)
    hh_ref[0] = h[:, :HF]
    hh_ref[1] = h[:, HF:]


def _tc_prescale(deg, x):
    return pl.pallas_call(
        _prescale_body,
        grid=(NP // BN,),
        in_specs=[pl.BlockSpec((BN, DW), lambda i: (i, 0)),
                  pl.BlockSpec((BN, D_IN), lambda i: (i, 0))],
        out_specs=pl.BlockSpec((2, BN, HF), lambda i: (0, i, 0)),
        out_shape=jax.ShapeDtypeStruct((2, NP, HF), jnp.bfloat16),
    )(deg, x)


def _mid_body(dd_ref, ds_ref, agg_ref, w1_ref, b1_ref, w2_ref, zh_ref):
    i = pl.program_id(0)
    a = agg_ref[pl.ds(i * BN, BN), :]
    b = agg_ref[pl.ds(NP + i * BN, BN), :]
    nd = _norm(dd_ref).astype(jnp.bfloat16)
    agg = jnp.concatenate([a, b], axis=1) * nd
    t = jnp.dot(agg, w1_ref[...], preferred_element_type=jnp.float32)
    t = jnp.maximum(t + b1_ref[...], 0.0)
    tb = (t * _norm(ds_ref)).astype(jnp.bfloat16)
    z = (jnp.dot(tb, w2_ref[...],
                 preferred_element_type=jnp.float32)).astype(jnp.bfloat16)
    zh_ref[0] = z[:, :HF]
    zh_ref[1] = z[:, HF:]


def _tc_mid(deg, agg1, w1, b1, w2):
    nb = NP // BN
    return pl.pallas_call(
        _mid_body,
        grid=(nb,),
        in_specs=[pl.BlockSpec((BN, DW), lambda i: (nb + i, 0)),
                  pl.BlockSpec((BN, DW), lambda i: (i, 0)),
                  pl.BlockSpec((2 * NP, HF), lambda i: (0, 0)),
                  pl.BlockSpec((D_IN, D_HID), lambda i: (0, 0)),
                  pl.BlockSpec((1, D_HID), lambda i: (0, 0)),
                  pl.BlockSpec((D_HID, D_OUT), lambda i: (0, 0))],
        out_specs=pl.BlockSpec((2, BN, HF), lambda i: (0, i, 0)),
        out_shape=jax.ShapeDtypeStruct((2, NP, HF), jnp.bfloat16),
    )(deg, deg, agg1, w1.astype(jnp.bfloat16), b1, w2.astype(jnp.bfloat16))


def _final_body(dd_ref, agg_ref, b2_ref, wp_ref, bp_ref, out_ref):
    i = pl.program_id(0)
    a = agg_ref[pl.ds(i * BN, BN), :]
    b = agg_ref[pl.ds(NP + i * BN, BN), :]
    agg = jnp.concatenate([a, b], axis=1).astype(jnp.float32) * _norm(dd_ref)
    x2 = jnp.maximum(agg + b2_ref[...], 0.0)
    logits = jnp.dot(x2, wp_ref[...], preferred_element_type=jnp.float32)
    out_ref[...] = jax.nn.sigmoid(logits + bp_ref[0, 0:1])


def _tc_final(deg, agg2, b2, wp, bp):
    nb = NP // BN
    return pl.pallas_call(
        _final_body,
        grid=(nb,),
        in_specs=[pl.BlockSpec((BN, DW), lambda i: (nb + i, 0)),
                  pl.BlockSpec((2 * NP, HF), lambda i: (0, 0)),
                  pl.BlockSpec((1, D_OUT), lambda i: (0, 0)),
                  pl.BlockSpec((D_OUT, 1), lambda i: (0, 0)),
                  pl.BlockSpec((1, 1), lambda i: (0, 0))],
        out_specs=pl.BlockSpec((BN, 1), lambda i: (i, 0)),
        out_shape=jax.ShapeDtypeStruct((NP, 1), jnp.float32),
    )(deg, agg2, b2, wp, bp)


def kernel(features, edge_index, edge_types, W1, b1, W2, b2, Wp, bp):
    er = edge_index.reshape(NW, NCH, K)
    ones16 = jnp.ones((K, DW), jnp.float32)
    zeros16 = jnp.zeros((RPT, DW), jnp.float32)
    zerosH = jnp.zeros((RPT, HF), jnp.bfloat16)

    deg = _sc_degrees(er, ones16, zeros16)
    hh = _tc_prescale(deg, features).reshape(2 * NP, HF)
    agg1 = _sc_aggregate(hh, er, zerosH)
    zh = _tc_mid(deg, agg1, W1, b1.reshape(1, D_HID), W2).reshape(2 * NP, HF)
    agg2 = _sc_aggregate(zh, er, zerosH)
    out = _tc_final(deg, agg2, b2.reshape(1, D_OUT), Wp, bp.reshape(1, 1))
    return out[:N, 0]
